# baseline (speedup 1.0000x reference)
import jax
import jax.numpy as jnp
from jax import lax
from jax.experimental import pallas as pl
from jax.experimental.pallas import tpu as pltpu
from jax.experimental.pallas import tpu_sc as plsc

N_VOTERS = 32
N_COLS = 1_000_000
LANES = 16
NUM_WORKERS = 32

SC_COLS = 409_600
PER_WORKER = SC_COLS // NUM_WORKERS
CHUNK = 1536
CHUNK_SIZES = [CHUNK] * (PER_WORKER // CHUNK) + (
    [PER_WORKER % CHUNK] if PER_WORKER % CHUNK else []
)
NCHUNKS = len(CHUNK_SIZES)
CHUNK_OFFS = [i * CHUNK for i in range(NCHUNKS)]

TC_BLOCK = 8192
TC_COLS = N_COLS - SC_COLS
TC_BLOCKS = -(-TC_COLS // TC_BLOCK)


def _sc_body(in_hbm, out_hbm, buf0, buf1, out_acc, isem0, isem1, osem):
    c = lax.axis_index("c")
    s = lax.axis_index("s")
    wid = s * 2 + c
    base = wid * PER_WORKER
    bufs = (buf0, buf1)
    isems = (isem0, isem1)

    def start_in(k):
        pltpu.async_copy(
            in_hbm.at[:, pl.ds(base + CHUNK_OFFS[k], CHUNK_SIZES[k])],
            bufs[k % 2].at[:, pl.ds(0, CHUNK_SIZES[k])],
            isems[k % 2],
        )

    def reduce_cols(src, out_off, n_cols):
        @plsc.parallel_loop(0, n_cols // LANES, unroll=2)
        def col_group(j):
            off = j * LANES
            vals = [src[i, pl.ds(off, LANES)] for i in range(N_VOTERS)]
            while len(vals) > 1:
                vals = [
                    vals[i] + vals[i + 1] for i in range(0, len(vals), 2)
                ]
            out_acc[pl.ds(out_off + off, LANES)] = jnp.where(
                vals[0] > N_VOTERS // 2, 1, 0
            ).astype(jnp.int32)

    start_in(0)
    for k in range(NCHUNKS):
        if k + 1 < NCHUNKS:
            start_in(k + 1)
        pltpu.make_async_copy(
            in_hbm.at[:, pl.ds(base + CHUNK_OFFS[k], CHUNK_SIZES[k])],
            bufs[k % 2].at[:, pl.ds(0, CHUNK_SIZES[k])],
            isems[k % 2],
        ).wait()
        reduce_cols(bufs[k % 2], CHUNK_OFFS[k], CHUNK_SIZES[k])

    pltpu.async_copy(out_acc, out_hbm.at[pl.ds(base, PER_WORKER)], osem)
    pltpu.make_async_copy(
        out_acc, out_hbm.at[pl.ds(base, PER_WORKER)], osem
    ).wait()


def _tc_body(x_ref, o_ref):
    o_ref[...] = (
        jnp.sum(x_ref[...], axis=0) > N_VOTERS // 2
    ).astype(jnp.int32)


@jax.jit
def _vote(inputs):
    sc = pl.kernel(
        _sc_body,
        out_type=jax.ShapeDtypeStruct((SC_COLS,), jnp.int32),
        mesh=plsc.VectorSubcoreMesh(core_axis_name="c", subcore_axis_name="s"),
        scratch_types=[
            pltpu.VMEM((N_VOTERS, CHUNK), jnp.int32),
            pltpu.VMEM((N_VOTERS, CHUNK), jnp.int32),
            pltpu.VMEM((PER_WORKER,), jnp.int32),
            pltpu.SemaphoreType.DMA,
            pltpu.SemaphoreType.DMA,
            pltpu.SemaphoreType.DMA,
        ],
    )
    out_sc = sc(inputs)
    out_tc = pl.pallas_call(
        _tc_body,
        grid=(TC_BLOCKS,),
        in_specs=[
            pl.BlockSpec(
                (N_VOTERS, TC_BLOCK), lambda j: (0, j + SC_COLS // TC_BLOCK)
            )
        ],
        out_specs=pl.BlockSpec((TC_BLOCK,), lambda j: (j,)),
        out_shape=jax.ShapeDtypeStruct((TC_COLS,), jnp.int32),
    )(inputs)
    return jnp.concatenate([out_sc, out_tc])


def kernel(inputs):
    return _vote(inputs)


# speedup vs baseline: 1.0804x; 1.0804x over previous
import jax
import jax.numpy as jnp
from jax import lax
from jax.experimental import pallas as pl
from jax.experimental.pallas import tpu as pltpu
from jax.experimental.pallas import tpu_sc as plsc

N_VOTERS = 32
N_COLS = 1_000_000
LANES = 16
NUM_WORKERS = 32
PER_WORKER = 31232
TAIL = N_COLS - NUM_WORKERS * PER_WORKER
TAIL_PAD = 640
TAIL_WORKER = 0
CHUNK = 1664
CHUNK_SIZES = [384] + [CHUNK] * 18 + [896]
assert sum(CHUNK_SIZES) == PER_WORKER and all(s % 128 == 0 for s in CHUNK_SIZES)
NCHUNKS = len(CHUNK_SIZES)
CHUNK_OFFS = [sum(CHUNK_SIZES[:i]) for i in range(NCHUNKS)]


def _body(
    in_hbm,
    tail_hbm,
    out_hbm,
    buf0,
    buf1,
    obuf0,
    obuf1,
    tail_buf,
    isem0,
    isem1,
    osem0,
    osem1,
    tsem,
):
    c = lax.axis_index("c")
    s = lax.axis_index("s")
    wid = s * 2 + c
    base = wid * PER_WORKER
    bufs = (buf0, buf1)
    obufs = (obuf0, obuf1)
    isems = (isem0, isem1)
    osems = (osem0, osem1)

    def in_copy(k):
        return pltpu.make_async_copy(
            in_hbm.at[:, pl.ds(base + CHUNK_OFFS[k], CHUNK_SIZES[k])],
            bufs[k % 2].at[:, pl.ds(0, CHUNK_SIZES[k])],
            isems[k % 2],
        )

    def out_copy(k):
        return pltpu.make_async_copy(
            obufs[k % 2].at[pl.ds(0, CHUNK_SIZES[k])],
            out_hbm.at[pl.ds(base + CHUNK_OFFS[k], CHUNK_SIZES[k])],
            osems[k % 2],
        )

    def reduce_cols(src, dst, n_cols):
        @plsc.parallel_loop(0, n_cols // LANES, unroll=1)
        def col_group(j):
            off = j * LANES
            vals = [src[i, pl.ds(off, LANES)] for i in range(N_VOTERS)]
            while len(vals) > 1:
                vals = [
                    vals[i] + vals[i + 1] for i in range(0, len(vals), 2)
                ]
            dst[pl.ds(off, LANES)] = jnp.where(
                vals[0] > N_VOTERS // 2, 1, 0
            ).astype(jnp.int32)

    @pl.when(wid == TAIL_WORKER)
    def _():
        pltpu.async_copy(tail_hbm, tail_buf, tsem)

    in_copy(0).start()
    in_copy(1).start()
    for k in range(NCHUNKS):
        in_copy(k).wait()
        if k >= 2:
            out_copy(k - 2).wait()
        reduce_cols(bufs[k % 2], obufs[k % 2], CHUNK_SIZES[k])
        out_copy(k).start()
        if k + 2 < NCHUNKS:
            in_copy(k + 2).start()
    out_copy(NCHUNKS - 2).wait()
    out_copy(NCHUNKS - 1).wait()

    @pl.when(wid == TAIL_WORKER)
    def _():
        tbase = NUM_WORKERS * PER_WORKER
        pltpu.make_async_copy(tail_hbm, tail_buf, tsem).wait()
        reduce_cols(tail_buf, obuf0, TAIL)
        pltpu.sync_copy(
            obuf0.at[pl.ds(0, TAIL)], out_hbm.at[pl.ds(tbase, TAIL)]
        )


@jax.jit
def _vote(inputs):
    tail = lax.slice(
        inputs, (0, NUM_WORKERS * PER_WORKER), (N_VOTERS, N_COLS)
    )
    tail = jnp.pad(tail, ((0, 0), (0, TAIL_PAD - TAIL)))
    k = pl.kernel(
        _body,
        out_type=jax.ShapeDtypeStruct((N_COLS,), jnp.int32),
        mesh=plsc.VectorSubcoreMesh(core_axis_name="c", subcore_axis_name="s"),
        scratch_types=[
            pltpu.VMEM((N_VOTERS, CHUNK), jnp.int32),
            pltpu.VMEM((N_VOTERS, CHUNK), jnp.int32),
            pltpu.VMEM((CHUNK,), jnp.int32),
            pltpu.VMEM((CHUNK,), jnp.int32),
            pltpu.VMEM((N_VOTERS, TAIL_PAD), jnp.int32),
            pltpu.SemaphoreType.DMA,
            pltpu.SemaphoreType.DMA,
            pltpu.SemaphoreType.DMA,
            pltpu.SemaphoreType.DMA,
            pltpu.SemaphoreType.DMA,
        ],
    )
    return k(inputs, tail)


def kernel(inputs):
    return _vote(inputs)


# speedup vs baseline: 1.1050x; 1.0228x over previous
import jax
import jax.numpy as jnp
from jax import lax
from jax.experimental import pallas as pl
from jax.experimental.pallas import tpu as pltpu
from jax.experimental.pallas import tpu_sc as plsc

N_VOTERS = 32
N_COLS = 1_000_000
LANES = 16
NUM_WORKERS = 32
PER_WORKER = 31232
TAIL = N_COLS - NUM_WORKERS * PER_WORKER
TAIL_PAD = 640
TAIL_WORKER = 0
NBUF = 3
CHUNK = 1280
CHUNK_SIZES = [384] + [CHUNK] * 24 + [128]
assert sum(CHUNK_SIZES) == PER_WORKER and all(s % 128 == 0 for s in CHUNK_SIZES)
NCHUNKS = len(CHUNK_SIZES)
CHUNK_OFFS = [sum(CHUNK_SIZES[:i]) for i in range(NCHUNKS)]
LAST_BUF2_K = max(k for k in range(NCHUNKS) if k % NBUF == 2)


def _body(
    in_hbm,
    tail_hbm,
    out_hbm,
    buf0,
    buf1,
    buf2,
    obuf0,
    obuf1,
    isem0,
    isem1,
    isem2,
    osem0,
    osem1,
    tsem,
):
    c = lax.axis_index("c")
    s = lax.axis_index("s")
    wid = s * 2 + c
    base = wid * PER_WORKER
    bufs = (buf0, buf1, buf2)
    obufs = (obuf0, obuf1)
    isems = (isem0, isem1, isem2)
    osems = (osem0, osem1)

    def in_copy(k):
        return pltpu.make_async_copy(
            in_hbm.at[:, pl.ds(base + CHUNK_OFFS[k], CHUNK_SIZES[k])],
            bufs[k % NBUF].at[:, pl.ds(0, CHUNK_SIZES[k])],
            isems[k % NBUF],
        )

    def out_copy(k):
        return pltpu.make_async_copy(
            obufs[k % 2].at[pl.ds(0, CHUNK_SIZES[k])],
            out_hbm.at[pl.ds(base + CHUNK_OFFS[k], CHUNK_SIZES[k])],
            osems[k % 2],
        )

    def reduce_cols(src, dst, n_cols):
        @plsc.parallel_loop(0, n_cols // LANES, unroll=1)
        def col_group(j):
            off = j * LANES
            vals = [src[i, pl.ds(off, LANES)] for i in range(N_VOTERS)]
            while len(vals) > 1:
                vals = [
                    vals[i] + vals[i + 1] for i in range(0, len(vals), 2)
                ]
            dst[pl.ds(off, LANES)] = jnp.where(
                vals[0] > N_VOTERS // 2, 1, 0
            ).astype(jnp.int32)

    for k in range(NBUF):
        in_copy(k).start()
    for k in range(NCHUNKS):
        in_copy(k).wait()
        if k >= 2:
            out_copy(k - 2).wait()
        reduce_cols(bufs[k % NBUF], obufs[k % 2], CHUNK_SIZES[k])
        out_copy(k).start()
        if k + NBUF < NCHUNKS:
            in_copy(k + NBUF).start()
        if k == LAST_BUF2_K:

            @pl.when(wid == TAIL_WORKER)
            def _():
                pltpu.async_copy(
                    tail_hbm, buf2.at[:, pl.ds(0, TAIL_PAD)], tsem
                )

    out_copy(NCHUNKS - 2).wait()
    out_copy(NCHUNKS - 1).wait()

    @pl.when(wid == TAIL_WORKER)
    def _():
        tbase = NUM_WORKERS * PER_WORKER
        pltpu.make_async_copy(
            tail_hbm, buf2.at[:, pl.ds(0, TAIL_PAD)], tsem
        ).wait()
        reduce_cols(buf2, obuf0, TAIL)
        pltpu.sync_copy(
            obuf0.at[pl.ds(0, TAIL)], out_hbm.at[pl.ds(tbase, TAIL)]
        )


@jax.jit
def _vote(inputs):
    tail = lax.slice(
        inputs, (0, NUM_WORKERS * PER_WORKER), (N_VOTERS, N_COLS)
    )
    tail = jnp.pad(tail, ((0, 0), (0, TAIL_PAD - TAIL)))
    k = pl.kernel(
        _body,
        out_type=jax.ShapeDtypeStruct((N_COLS,), jnp.int32),
        mesh=plsc.VectorSubcoreMesh(core_axis_name="c", subcore_axis_name="s"),
        scratch_types=[
            pltpu.VMEM((N_VOTERS, CHUNK), jnp.int32),
            pltpu.VMEM((N_VOTERS, CHUNK), jnp.int32),
            pltpu.VMEM((N_VOTERS, CHUNK), jnp.int32),
            pltpu.VMEM((CHUNK,), jnp.int32),
            pltpu.VMEM((CHUNK,), jnp.int32),
            pltpu.SemaphoreType.DMA,
            pltpu.SemaphoreType.DMA,
            pltpu.SemaphoreType.DMA,
            pltpu.SemaphoreType.DMA,
            pltpu.SemaphoreType.DMA,
            pltpu.SemaphoreType.DMA,
        ],
    )
    return k(inputs, tail)


def kernel(inputs):
    return _vote(inputs)


# speedup vs baseline: 1.1792x; 1.0672x over previous
import jax
import jax.numpy as jnp
from jax import lax
from jax.experimental import pallas as pl
from jax.experimental.pallas import tpu as pltpu
from jax.experimental.pallas import tpu_sc as plsc

N_VOTERS = 32
N_COLS = 1_000_000
LANES = 16
NUM_WORKERS = 32
PER_WORKER = 19200
NBUF = 3
CHUNK = 1280
CHUNK_SIZES = [384] + [CHUNK] * 14 + [896]
assert sum(CHUNK_SIZES) == PER_WORKER and all(s % 128 == 0 for s in CHUNK_SIZES)
NCHUNKS = len(CHUNK_SIZES)
CHUNK_OFFS = [sum(CHUNK_SIZES[:i]) for i in range(NCHUNKS)]
SC_COLS = NUM_WORKERS * PER_WORKER
TC_BLOCK = 8192
TC_COLS = N_COLS - SC_COLS
TC_BLOCKS = -(-TC_COLS // TC_BLOCK)


def _body(
    in_hbm,
    out_hbm,
    buf0,
    buf1,
    buf2,
    obuf0,
    obuf1,
    isem0,
    isem1,
    isem2,
    osem0,
    osem1,
):
    c = lax.axis_index("c")
    s = lax.axis_index("s")
    wid = s * 2 + c
    base = wid * PER_WORKER
    bufs = (buf0, buf1, buf2)
    obufs = (obuf0, obuf1)
    isems = (isem0, isem1, isem2)
    osems = (osem0, osem1)

    def in_copy(k):
        return pltpu.make_async_copy(
            in_hbm.at[:, pl.ds(base + CHUNK_OFFS[k], CHUNK_SIZES[k])],
            bufs[k % NBUF].at[:, pl.ds(0, CHUNK_SIZES[k])],
            isems[k % NBUF],
        )

    def out_copy(k):
        return pltpu.make_async_copy(
            obufs[k % 2].at[pl.ds(0, CHUNK_SIZES[k])],
            out_hbm.at[pl.ds(base + CHUNK_OFFS[k], CHUNK_SIZES[k])],
            osems[k % 2],
        )

    def reduce_cols(src, dst, n_cols):
        @plsc.parallel_loop(0, n_cols // LANES, unroll=1)
        def col_group(j):
            off = j * LANES
            vals = [src[i, pl.ds(off, LANES)] for i in range(N_VOTERS)]
            while len(vals) > 1:
                vals = [
                    vals[i] + vals[i + 1] for i in range(0, len(vals), 2)
                ]
            dst[pl.ds(off, LANES)] = jnp.where(
                vals[0] > N_VOTERS // 2, 1, 0
            ).astype(jnp.int32)

    for k in range(NBUF):
        in_copy(k).start()
    for k in range(NCHUNKS):
        in_copy(k).wait()
        if k >= 2:
            out_copy(k - 2).wait()
        reduce_cols(bufs[k % NBUF], obufs[k % 2], CHUNK_SIZES[k])
        out_copy(k).start()
        if k + NBUF < NCHUNKS:
            in_copy(k + NBUF).start()
    out_copy(NCHUNKS - 2).wait()
    out_copy(NCHUNKS - 1).wait()


def _tc_body(x_ref, o_ref):
    o_ref[...] = (
        jnp.sum(x_ref[...], axis=0) > N_VOTERS // 2
    ).astype(jnp.int32)


@jax.jit
def _vote(inputs):
    out_tc = pl.pallas_call(
        _tc_body,
        grid=(TC_BLOCKS,),
        in_specs=[
            pl.BlockSpec(
                (N_VOTERS, TC_BLOCK), lambda j: (0, j + SC_COLS // TC_BLOCK)
            )
        ],
        out_specs=pl.BlockSpec((TC_BLOCK,), lambda j: (j,)),
        out_shape=jax.ShapeDtypeStruct((TC_COLS,), jnp.int32),
    )(inputs)
    k = pl.kernel(
        _body,
        out_type=jax.ShapeDtypeStruct((SC_COLS,), jnp.int32),
        mesh=plsc.VectorSubcoreMesh(core_axis_name="c", subcore_axis_name="s"),
        scratch_types=[
            pltpu.VMEM((N_VOTERS, CHUNK), jnp.int32),
            pltpu.VMEM((N_VOTERS, CHUNK), jnp.int32),
            pltpu.VMEM((N_VOTERS, CHUNK), jnp.int32),
            pltpu.VMEM((CHUNK,), jnp.int32),
            pltpu.VMEM((CHUNK,), jnp.int32),
            pltpu.SemaphoreType.DMA,
            pltpu.SemaphoreType.DMA,
            pltpu.SemaphoreType.DMA,
            pltpu.SemaphoreType.DMA,
            pltpu.SemaphoreType.DMA,
        ],
    )
    out_sc = k(inputs)
    return jnp.concatenate([out_sc, out_tc])


def kernel(inputs):
    return _vote(inputs)
